# baseline (speedup 1.0000x reference)
import functools
import math

import jax
import jax.numpy as jnp
import numpy as np
from jax import lax
from jax.experimental import pallas as pl
from jax.experimental.pallas import tpu as pltpu
from jax.experimental.pallas import tpu_sc as plsc

E = 4
B = 1024
C = 256
BT = 32
NBLK = 40
B_PAD = NBLK * BT
NITER = 15
_F32 = jnp.float32
_I32 = jnp.int32


def _conv2d(x, w, b):
    y = jax.lax.conv_general_dilated(x, w, window_strides=(1, 1), padding='SAME',
                                     dimension_numbers=('NCHW', 'OIHW', 'NCHW'))
    return y + b[None, :, None, None]


def _avgpool(x, k):
    return jax.lax.reduce_window(x, 0.0, jax.lax.add, (1, 1, k, k), (1, 1, k, k), 'VALID') / float(k * k)


def _scorer_trunk(p, e, x):
    h = jax.nn.relu(_conv2d(x, p[f'e{e}_sc_w'], p[f'e{e}_sc_b']))
    k = h.shape[2] // 4
    h = _avgpool(h, k)
    h = h.reshape(h.shape[0], -1)
    h = jax.nn.relu(h @ p[f'e{e}_sfc_w'] + p[f'e{e}_sfc_b'])
    return h


def _scores_noisy(x, params, targets):
    Bn = x.shape[0]
    feats = [_scorer_trunk(params, e, x) for e in range(E)]
    scores = jnp.stack([(feats[e] @ params[f'e{e}_sh_w'] + params[f'e{e}_sh_b'])[:, 0]
                        for e in range(E)], axis=1)
    class_logits = jnp.stack([feats[e] @ params[f'e{e}_scl_w'] + params[f'e{e}_scl_b']
                              for e in range(E)], axis=1)
    class_probs = jax.nn.softmax(class_logits, axis=2)
    tgt = jnp.broadcast_to(targets[:, None].astype(jnp.int32), (Bn, E))
    gt_probs = jnp.take_along_axis(class_probs, tgt[:, :, None], axis=2)[:, :, 0]
    combined = scores + 1.0 * jnp.log(jnp.clip(gt_probs, 1e-9, None))
    return combined / 0.1


def _waterfall_body(sn_ref, assign_ref, slot_ref, perm_ref, bexp_ref):
    sn = sn_ref[:]
    f32, i32 = _F32, _I32

    def iota_f32(shape, dim):
        return jax.lax.broadcasted_iota(i32, shape, dim).astype(f32)

    rj = jax.lax.broadcasted_iota(i32, (B, B), 0)
    ci = jax.lax.broadcasted_iota(i32, (B, B), 1)
    ltt = (rj <= ci).astype(f32)

    iota_e = iota_f32((E, B), 0)
    iota_e1 = iota_f32((E, 1), 0)

    assign = jnp.zeros((E, B), f32)
    for it in range(NITER):
        cap = jnp.sum(assign, axis=1, keepdims=True)
        rem = 1.0 - jnp.sum(assign, axis=0, keepdims=True)
        deficit = jnp.clip(cap * (1.0 / C), 0.0, 1.0)
        s = sn - deficit
        s = jnp.where(cap >= C, -1e30, s)
        m = jnp.max(s, axis=0, keepdims=True)
        cand = jnp.where(s == m, iota_e, float(E))
        sel = jnp.min(cand, axis=0, keepdims=True)
        onehot = (iota_e == sel).astype(f32)
        want = onehot * rem
        rank = jnp.dot(want, ltt, preferred_element_type=f32)
        space = jnp.minimum(C - cap, float(2 ** it))
        take = want * (rank <= space).astype(f32)
        assign = assign + take

    cap = jnp.sum(assign, axis=1, keepdims=True)
    rem = 1.0 - jnp.sum(assign, axis=0, keepdims=True)
    mn = jnp.min(cap, axis=0, keepdims=True)
    cand = jnp.where(cap == mn, iota_e1, float(E))
    least = jnp.min(cand, axis=0, keepdims=True)
    assign = assign + (iota_e1 == least).astype(f32) * rem
    assign_ref[:] = assign

    count = jnp.sum(assign, axis=1, keepdims=True)
    pc = jnp.floor((count + (BT - 1)) * (1.0 / BT)) * BT
    slt4 = (jax.lax.broadcasted_iota(i32, (E, E), 0) >
            jax.lax.broadcasted_iota(i32, (E, E), 1)).astype(f32)
    starts = jnp.dot(slt4, pc, preferred_element_type=f32)
    rank_all = jnp.dot(assign, ltt, preferred_element_type=f32)
    slotv = jnp.sum(assign * (starts + rank_all - 1.0), axis=0, keepdims=True)
    slot_ref[:] = slotv.astype(i32)

    oh = (iota_f32((B_PAD, B), 0) == slotv).astype(f32)
    idx_col = iota_f32((B, 1), 0)
    perm = jnp.dot(oh, idx_col, preferred_element_type=f32)
    perm_ref[:] = perm.astype(i32)

    bstart = starts * (1.0 / BT)
    iota_g = iota_f32((E, NBLK), 1)
    bexp = jnp.sum((iota_g >= bstart).astype(f32), axis=0, keepdims=True) - 1.0
    bexp_ref[:] = bexp.astype(i32)


def _waterfall_route(sn_t):
    return pl.pallas_call(
        _waterfall_body,
        out_shape=(
            jax.ShapeDtypeStruct((E, B), _F32),
            jax.ShapeDtypeStruct((1, B), _I32),
            jax.ShapeDtypeStruct((B_PAD, 1), _I32),
            jax.ShapeDtypeStruct((1, NBLK), _I32),
        ),
    )(sn_t)


def _sc_gather(table, idx):
    nrows, d = idx.shape[0], table.shape[1]
    info = plsc.get_sparse_core_info()
    nw = info.num_cores * info.num_subcores
    bw = nrows // nw
    mesh = plsc.VectorSubcoreMesh(core_axis_name="c", subcore_axis_name="s")

    @functools.partial(
        pl.kernel, mesh=mesh,
        out_type=jax.ShapeDtypeStruct((nrows, d), _F32),
        scratch_types=[
            pltpu.VMEM((bw,), _I32),
            pltpu.VMEM((bw, d), _F32),
            pltpu.SemaphoreType.DMA,
        ],
    )
    def gat(table_hbm, idx_hbm, out_hbm, idx_v, rows_v, sem):
        wid = lax.axis_index("s") * info.num_cores + lax.axis_index("c")
        base = wid * bw
        pltpu.sync_copy(idx_hbm.at[pl.ds(base, bw)], idx_v)
        pltpu.async_copy(table_hbm.at[idx_v], rows_v, sem).wait()
        pltpu.sync_copy(rows_v, out_hbm.at[pl.ds(base, bw)])

    return gat(table, idx)


def _z(*shape):
    return jnp.zeros(shape, _F32)


def _conv9(xf, w, hp, cin):
    off0 = hp + 1
    nv = hp * hp - 2 * off0
    ne = nv + 2
    parts = [xf[off0 + o - 1:off0 + o - 1 + ne, :] for o in (-hp, 0, hp)]
    xcat = jnp.concatenate(parts, axis=1)
    k3 = 3 * cin
    p0 = jnp.dot(xcat, w[0:k3, :], preferred_element_type=_F32)
    p1 = jnp.dot(xcat, w[k3:2 * k3, :], preferred_element_type=_F32)
    p2 = jnp.dot(xcat, w[2 * k3:3 * k3, :], preferred_element_type=_F32)
    return p0[0:nv, :] + p1[1:1 + nv, :] + p2[2:2 + nv, :]


def _convpool_val(xf, w, bias, hp, cin, cout):
    off0 = hp + 1
    h = hp - 2
    hh = h // 2
    y = jnp.maximum(_conv9(xf, w, hp, cin) + bias, 0.0)
    full = jnp.concatenate([_z(off0, cout), y, _z(off0, cout)], axis=0)
    g = full.reshape(hp, hp, cout)[1:1 + h, 1:1 + h, :]
    g = g.reshape(hh, 2, hh, 2, cout)
    p = jnp.max(jnp.max(g, axis=3), axis=1)
    p = jnp.concatenate([_z(hh, 1, cout), p, _z(hh, 1, cout)], axis=1)
    p = jnp.concatenate([_z(1, hh + 2, cout), p, _z(1, hh + 2, cout)], axis=0)
    return p.reshape((hh + 2) * (hh + 2), cout)


def _enc12_body(bexp_ref, in_ref, w1_ref, b1_ref, w2_ref, b2_ref, out_ref):
    w1, b1 = w1_ref[0], b1_ref[0]
    w2, b2 = w2_ref[0], b2_ref[0]
    r = jax.lax.broadcasted_iota(_I32, (1086, 1), 0) + 35
    rm = jax.lax.rem(r, 34)
    mask1 = ((rm != 0) & (rm != 33)).astype(_F32)

    def tok(t, carry):
        xt = in_ref[t].reshape(32, 32, 3)
        xt = jnp.concatenate([_z(32, 1, 3), xt, _z(32, 1, 3)], axis=1)
        xt = jnp.concatenate([_z(1, 34, 3), xt, _z(1, 34, 3)], axis=0)
        xf = xt.reshape(1156, 3)
        y1 = jnp.maximum(_conv9(xf, w1, 34, 3) + b1, 0.0) * mask1
        y1f = jnp.concatenate([_z(35, 64), y1, _z(35, 64)], axis=0)
        out_ref[t] = _convpool_val(y1f, w2, b2, 34, 64, 64)
        return carry

    jax.lax.fori_loop(0, BT, tok, 0)


def _enc34_body(bexp_ref, in_ref, w3_ref, b3_ref, w4_ref, b4_ref,
                prw_ref, prb_ref, clw_ref, clb_ref, out_ref, fe_ref):
    w3, b3 = w3_ref[0], b3_ref[0]
    w4, b4 = w4_ref[0], b4_ref[0]
    r = jax.lax.broadcasted_iota(_I32, (78, 1), 0) + 11
    rm = jax.lax.rem(r, 10)
    mask4 = ((rm >= 1) & (rm <= 8)).astype(_F32)

    def tok(t, carry):
        p3 = _convpool_val(in_ref[t], w3, b3, 18, 64, 128)
        y4 = jnp.maximum(_conv9(p3, w4, 10, 128) + b4, 0.0)
        fe = jnp.sum(y4 * mask4, axis=0, keepdims=True) * (1.0 / 64.0)
        fe_ref[pl.ds(t, 1), :] = fe
        return carry

    jax.lax.fori_loop(0, BT, tok, 0)
    feats = fe_ref[:]
    z = jnp.dot(feats, prw_ref[0], preferred_element_type=_F32) + prb_ref[0]
    out_ref[:] = jnp.dot(z, clw_ref[0], preferred_element_type=_F32) + clb_ref[0]


def _expert_spec(k, o):
    return pl.BlockSpec((1, k, o), lambda g, b: (b[g], 0, 0))


def _tok_spec(s, c):
    return pl.BlockSpec((BT, s, c), lambda g, b: (g, 0, 0))


def _run_enc12(bexp, xs, w1, b1, w2, b2):
    gs = pltpu.PrefetchScalarGridSpec(
        num_scalar_prefetch=1, grid=(NBLK,),
        in_specs=[_tok_spec(1024, 3), _expert_spec(27, 64), _expert_spec(1, 64),
                  _expert_spec(576, 64), _expert_spec(1, 64)],
        out_specs=_tok_spec(324, 64),
    )
    return pl.pallas_call(_enc12_body, grid_spec=gs,
                          out_shape=jax.ShapeDtypeStruct((B_PAD, 324, 64), _F32),
                          )(bexp, xs, w1, b1, w2, b2)


def _run_enc34(bexp, a, w3, b3, w4, b4, prw, prb, clw, clb):
    gs = pltpu.PrefetchScalarGridSpec(
        num_scalar_prefetch=1, grid=(NBLK,),
        in_specs=[_tok_spec(324, 64), _expert_spec(576, 128), _expert_spec(1, 128),
                  _expert_spec(1152, 256), _expert_spec(1, 256),
                  _expert_spec(256, 256), _expert_spec(1, 256),
                  _expert_spec(256, 128), _expert_spec(1, 128)],
        out_specs=pl.BlockSpec((BT, 128), lambda g, b: (g, 0)),
        scratch_shapes=[pltpu.VMEM((BT, 256), _F32)],
    )
    return pl.pallas_call(_enc34_body, grid_spec=gs,
                          out_shape=jax.ShapeDtypeStruct((B_PAD, 128), _F32),
                          )(bexp, a, w3, b3, w4, b4, prw, prb, clw, clb)


def _fold_conv(params, layer, cin, cout):
    s = 1.0 / np.sqrt(1.0 + 1e-5)
    ws, bs = [], []
    for e in range(E):
        wv = params[f'e{e}_c{layer}_w']
        bv = params[f'e{e}_c{layer}_b']
        g = params[f'e{e}_bn{layer}_g'] * s
        bb = params[f'e{e}_bn{layer}_b']
        wim = jnp.transpose(wv, (3, 2, 1, 0)) * g[None, None, None, :]
        ws.append(wim.reshape(9 * cin, cout))
        bs.append((bv * g + bb)[None, :])
    return jnp.stack(ws), jnp.stack(bs)


def kernel(x, params, targets):
    sn = _scores_noisy(x, params, targets)
    assign, slot, perm, bexp = _waterfall_route(sn.T)
    del assign

    x2d = jnp.transpose(x, (0, 2, 3, 1)).reshape(B, 32 * 32 * 3)
    xs = _sc_gather(x2d, perm.reshape(B_PAD))

    bexp_f = bexp.reshape(NBLK)
    w1, b1 = _fold_conv(params, 1, 3, 64)
    w2, b2 = _fold_conv(params, 2, 64, 64)
    w3, b3 = _fold_conv(params, 3, 64, 128)
    w4, b4 = _fold_conv(params, 4, 128, 256)
    prw = jnp.stack([params[f'e{e}_pr_w'] for e in range(E)])
    prb = jnp.stack([params[f'e{e}_pr_b'][None, :] for e in range(E)])
    clw = jnp.stack([jnp.pad(params[f'e{e}_cl_w'], ((0, 0), (0, 118))) for e in range(E)])
    clb = jnp.stack([jnp.pad(params[f'e{e}_cl_b'], (0, 118))[None, :] for e in range(E)])

    a2 = _run_enc12(bexp_f, xs.reshape(B_PAD, 1024, 3), w1, b1, w2, b2)
    lg = _run_enc34(bexp_f, a2, w3, b3, w4, b4, prw, prb, clw, clb)

    out128 = _sc_gather(lg, slot.reshape(B))
    return out128[:, :10]


# speedup vs baseline: 1.0798x; 1.0798x over previous
import functools
import math

import jax
import jax.numpy as jnp
import numpy as np
from jax import lax
from jax.experimental import pallas as pl
from jax.experimental.pallas import tpu as pltpu
from jax.experimental.pallas import tpu_sc as plsc

E = 4
B = 1024
C = 256
BT = 32
NBLK = 40
B_PAD = NBLK * BT
NITER = 15
_F32 = jnp.float32
_I32 = jnp.int32


def _conv2d(x, w, b):
    y = jax.lax.conv_general_dilated(x, w, window_strides=(1, 1), padding='SAME',
                                     dimension_numbers=('NCHW', 'OIHW', 'NCHW'))
    return y + b[None, :, None, None]


def _avgpool(x, k):
    return jax.lax.reduce_window(x, 0.0, jax.lax.add, (1, 1, k, k), (1, 1, k, k), 'VALID') / float(k * k)


def _scorer_trunk(p, e, x):
    h = jax.nn.relu(_conv2d(x, p[f'e{e}_sc_w'], p[f'e{e}_sc_b']))
    k = h.shape[2] // 4
    h = _avgpool(h, k)
    h = h.reshape(h.shape[0], -1)
    h = jax.nn.relu(h @ p[f'e{e}_sfc_w'] + p[f'e{e}_sfc_b'])
    return h


def _scores_noisy(x, params, targets):
    Bn = x.shape[0]
    feats = [_scorer_trunk(params, e, x) for e in range(E)]
    scores = jnp.stack([(feats[e] @ params[f'e{e}_sh_w'] + params[f'e{e}_sh_b'])[:, 0]
                        for e in range(E)], axis=1)
    class_logits = jnp.stack([feats[e] @ params[f'e{e}_scl_w'] + params[f'e{e}_scl_b']
                              for e in range(E)], axis=1)
    class_probs = jax.nn.softmax(class_logits, axis=2)
    tgt = jnp.broadcast_to(targets[:, None].astype(jnp.int32), (Bn, E))
    gt_probs = jnp.take_along_axis(class_probs, tgt[:, :, None], axis=2)[:, :, 0]
    combined = scores + 1.0 * jnp.log(jnp.clip(gt_probs, 1e-9, None))
    return combined / 0.1


def _waterfall_body(sn_ref, assign_ref, slot_ref, perm_ref, bexp_ref):
    sn = sn_ref[:]
    f32, i32 = _F32, _I32

    def iota_f32(shape, dim):
        return jax.lax.broadcasted_iota(i32, shape, dim).astype(f32)

    rj = jax.lax.broadcasted_iota(i32, (B, B), 0)
    ci = jax.lax.broadcasted_iota(i32, (B, B), 1)
    ltt = (rj <= ci).astype(f32)

    iota_e = iota_f32((E, B), 0)
    iota_e1 = iota_f32((E, 1), 0)

    assign = jnp.zeros((E, B), f32)
    for it in range(NITER):
        cap = jnp.sum(assign, axis=1, keepdims=True)
        rem = 1.0 - jnp.sum(assign, axis=0, keepdims=True)
        deficit = jnp.clip(cap * (1.0 / C), 0.0, 1.0)
        s = sn - deficit
        s = jnp.where(cap >= C, -1e30, s)
        m = jnp.max(s, axis=0, keepdims=True)
        cand = jnp.where(s == m, iota_e, float(E))
        sel = jnp.min(cand, axis=0, keepdims=True)
        onehot = (iota_e == sel).astype(f32)
        want = onehot * rem
        rank = jnp.dot(want, ltt, preferred_element_type=f32)
        space = jnp.minimum(C - cap, float(2 ** it))
        take = want * (rank <= space).astype(f32)
        assign = assign + take

    cap = jnp.sum(assign, axis=1, keepdims=True)
    rem = 1.0 - jnp.sum(assign, axis=0, keepdims=True)
    mn = jnp.min(cap, axis=0, keepdims=True)
    cand = jnp.where(cap == mn, iota_e1, float(E))
    least = jnp.min(cand, axis=0, keepdims=True)
    assign = assign + (iota_e1 == least).astype(f32) * rem
    assign_ref[:] = assign

    count = jnp.sum(assign, axis=1, keepdims=True)
    pc = jnp.floor((count + (BT - 1)) * (1.0 / BT)) * BT
    slt4 = (jax.lax.broadcasted_iota(i32, (E, E), 0) >
            jax.lax.broadcasted_iota(i32, (E, E), 1)).astype(f32)
    starts = jnp.dot(slt4, pc, preferred_element_type=f32)
    rank_all = jnp.dot(assign, ltt, preferred_element_type=f32)
    slotv = jnp.sum(assign * (starts + rank_all - 1.0), axis=0, keepdims=True)
    slot_ref[:] = slotv.astype(i32)

    oh = (iota_f32((B_PAD, B), 0) == slotv).astype(f32)
    idx_col = iota_f32((B, 1), 0)
    perm = jnp.dot(oh, idx_col, preferred_element_type=f32)
    perm_ref[:] = perm.astype(i32)

    bstart = starts * (1.0 / BT)
    iota_g = iota_f32((E, NBLK), 1)
    bexp = jnp.sum((iota_g >= bstart).astype(f32), axis=0, keepdims=True) - 1.0
    bexp_ref[:] = bexp.astype(i32)


def _waterfall_route(sn_t):
    return pl.pallas_call(
        _waterfall_body,
        out_shape=(
            jax.ShapeDtypeStruct((E, B), _F32),
            jax.ShapeDtypeStruct((1, B), _I32),
            jax.ShapeDtypeStruct((B_PAD, 1), _I32),
            jax.ShapeDtypeStruct((1, NBLK), _I32),
        ),
    )(sn_t)


def _sc_gather(table, idx):
    nrows, d = idx.shape[0], table.shape[1]
    info = plsc.get_sparse_core_info()
    nw = info.num_cores * info.num_subcores
    bw = nrows // nw
    mesh = plsc.VectorSubcoreMesh(core_axis_name="c", subcore_axis_name="s")

    @functools.partial(
        pl.kernel, mesh=mesh,
        out_type=jax.ShapeDtypeStruct((nrows, d), _F32),
        scratch_types=[
            pltpu.VMEM((bw,), _I32),
            pltpu.VMEM((bw, d), _F32),
            pltpu.SemaphoreType.DMA,
        ],
    )
    def gat(table_hbm, idx_hbm, out_hbm, idx_v, rows_v, sem):
        wid = lax.axis_index("s") * info.num_cores + lax.axis_index("c")
        base = wid * bw
        pltpu.sync_copy(idx_hbm.at[pl.ds(base, bw)], idx_v)
        pltpu.async_copy(table_hbm.at[idx_v], rows_v, sem).wait()
        pltpu.sync_copy(rows_v, out_hbm.at[pl.ds(base, bw)])

    return gat(table, idx)


def _z(*shape):
    return jnp.zeros(shape, _F32)


def _conv9(xf, w, hp, cin):
    S = xf.shape[0]
    off0 = hp + 1
    nv = S - 2 * off0
    ne = nv + 2
    parts = [xf[off0 + o - 1:off0 + o - 1 + ne, :] for o in (-hp, 0, hp)]
    xcat = jnp.concatenate(parts, axis=1)
    k3 = 3 * cin
    p0 = jnp.dot(xcat, w[0:k3, :], preferred_element_type=_F32)
    p1 = jnp.dot(xcat, w[k3:2 * k3, :], preferred_element_type=_F32)
    p2 = jnp.dot(xcat, w[2 * k3:3 * k3, :], preferred_element_type=_F32)
    return p0[0:nv, :] + p1[1:1 + nv, :] + p2[2:2 + nv, :]


def _gridmask(nv, off0, stride, hp, lo, hi):
    r = jax.lax.broadcasted_iota(_I32, (nv, 1), 0) + off0
    qt = jax.lax.rem(r, stride)
    cm = jax.lax.rem(qt, hp)
    return ((qt >= lo) & (qt <= hi) & (cm != 0) & (cm != hp - 1)).astype(_F32)


def _enc12_body(bexp_ref, in_ref, w1_ref, b1_ref, w2_ref, b2_ref, out_ref):
    G = 4
    S = G * 1156
    w1, b1 = w1_ref[0], b1_ref[0]
    w2, b2 = w2_ref[0], b2_ref[0]
    m1 = _gridmask(S - 70, 35, 1156, 34, 35, 1120)

    def grp(i, carry):
        x4 = in_ref[pl.ds(G * i, G)].reshape(G, 32, 32, 3)
        x4 = jnp.concatenate([_z(G, 32, 1, 3), x4, _z(G, 32, 1, 3)], axis=2)
        x4 = jnp.concatenate([_z(G, 1, 34, 3), x4, _z(G, 1, 34, 3)], axis=1)
        xf = x4.reshape(S, 3)
        y1 = jnp.maximum(_conv9(xf, w1, 34, 3) + b1, 0.0) * m1
        y1f = jnp.concatenate([_z(35, 64), y1, _z(35, 64)], axis=0)
        y2 = jnp.maximum(_conv9(y1f, w2, 34, 64) + b2, 0.0) * m1
        y2f = jnp.concatenate([_z(35, 64), y2, _z(35, 64)], axis=0)
        g = y2f.reshape(G, 34, 34, 64)[:, 1:33, 1:33, :].reshape(G, 16, 2, 16, 2, 64)
        p = jnp.max(jnp.max(g, axis=4), axis=2)
        p = jnp.concatenate([_z(G, 16, 1, 64), p, _z(G, 16, 1, 64)], axis=2)
        p = jnp.concatenate([_z(G, 1, 18, 64), p, _z(G, 1, 18, 64)], axis=1)
        out_ref[pl.ds(G * i, G)] = p.reshape(G, 324, 64)
        return carry

    jax.lax.fori_loop(0, BT // G, grp, 0)


def _enc34_body(bexp_ref, in_ref, w3_ref, b3_ref, w4_ref, b4_ref,
                prw_ref, prb_ref, clw_ref, clb_ref, out_ref, fe_ref):
    G = 4
    S3 = G * 324
    S4 = G * 100
    w3, b3 = w3_ref[0], b3_ref[0]
    w4, b4 = w4_ref[0], b4_ref[0]
    m3 = _gridmask(S3 - 38, 19, 324, 18, 19, 304)
    m4 = _gridmask(S4 - 22, 11, 100, 10, 11, 88)

    def grp(i, carry):
        a4 = in_ref[pl.ds(G * i, G)].reshape(S3, 64)
        y3 = jnp.maximum(_conv9(a4, w3, 18, 64) + b3, 0.0) * m3
        y3f = jnp.concatenate([_z(19, 128), y3, _z(19, 128)], axis=0)
        g = y3f.reshape(G, 18, 18, 128)[:, 1:17, 1:17, :].reshape(G, 8, 2, 8, 2, 128)
        p = jnp.max(jnp.max(g, axis=4), axis=2)
        p = jnp.concatenate([_z(G, 8, 1, 128), p, _z(G, 8, 1, 128)], axis=2)
        p = jnp.concatenate([_z(G, 1, 10, 128), p, _z(G, 1, 10, 128)], axis=1)
        y4 = jnp.maximum(_conv9(p.reshape(S4, 128), w4, 10, 128) + b4, 0.0) * m4
        for k in range(G):
            fe = jnp.sum(y4[100 * k:100 * k + 78, :], axis=0, keepdims=True) * (1.0 / 64.0)
            fe_ref[pl.ds(G * i + k, 1), :] = fe
        return carry

    jax.lax.fori_loop(0, BT // G, grp, 0)
    feats = fe_ref[:]
    z = jnp.dot(feats, prw_ref[0], preferred_element_type=_F32) + prb_ref[0]
    out_ref[:] = jnp.dot(z, clw_ref[0], preferred_element_type=_F32) + clb_ref[0]


def _expert_spec(k, o):
    return pl.BlockSpec((1, k, o), lambda g, b: (b[g], 0, 0))


def _tok_spec(s, c):
    return pl.BlockSpec((BT, s, c), lambda g, b: (g, 0, 0))


def _run_enc12(bexp, xs, w1, b1, w2, b2):
    gs = pltpu.PrefetchScalarGridSpec(
        num_scalar_prefetch=1, grid=(NBLK,),
        in_specs=[_tok_spec(1024, 3), _expert_spec(27, 64), _expert_spec(1, 64),
                  _expert_spec(576, 64), _expert_spec(1, 64)],
        out_specs=_tok_spec(324, 64),
    )
    return pl.pallas_call(_enc12_body, grid_spec=gs,
                          out_shape=jax.ShapeDtypeStruct((B_PAD, 324, 64), _F32),
                          )(bexp, xs, w1, b1, w2, b2)


def _run_enc34(bexp, a, w3, b3, w4, b4, prw, prb, clw, clb):
    gs = pltpu.PrefetchScalarGridSpec(
        num_scalar_prefetch=1, grid=(NBLK,),
        in_specs=[_tok_spec(324, 64), _expert_spec(576, 128), _expert_spec(1, 128),
                  _expert_spec(1152, 256), _expert_spec(1, 256),
                  _expert_spec(256, 256), _expert_spec(1, 256),
                  _expert_spec(256, 128), _expert_spec(1, 128)],
        out_specs=pl.BlockSpec((BT, 128), lambda g, b: (g, 0)),
        scratch_shapes=[pltpu.VMEM((BT, 256), _F32)],
    )
    return pl.pallas_call(_enc34_body, grid_spec=gs,
                          out_shape=jax.ShapeDtypeStruct((B_PAD, 128), _F32),
                          )(bexp, a, w3, b3, w4, b4, prw, prb, clw, clb)


def _fold_conv(params, layer, cin, cout):
    s = 1.0 / np.sqrt(1.0 + 1e-5)
    ws, bs = [], []
    for e in range(E):
        wv = params[f'e{e}_c{layer}_w']
        bv = params[f'e{e}_c{layer}_b']
        g = params[f'e{e}_bn{layer}_g'] * s
        bb = params[f'e{e}_bn{layer}_b']
        wim = jnp.transpose(wv, (3, 2, 1, 0)) * g[None, None, None, :]
        ws.append(wim.reshape(9 * cin, cout))
        bs.append((bv * g + bb)[None, :])
    return jnp.stack(ws), jnp.stack(bs)


def kernel(x, params, targets):
    sn = _scores_noisy(x, params, targets)
    assign, slot, perm, bexp = _waterfall_route(sn.T)
    del assign

    x2d = jnp.transpose(x, (0, 2, 3, 1)).reshape(B, 32 * 32 * 3)
    xs = _sc_gather(x2d, perm.reshape(B_PAD))

    bexp_f = bexp.reshape(NBLK)
    w1, b1 = _fold_conv(params, 1, 3, 64)
    w2, b2 = _fold_conv(params, 2, 64, 64)
    w3, b3 = _fold_conv(params, 3, 64, 128)
    w4, b4 = _fold_conv(params, 4, 128, 256)
    prw = jnp.stack([params[f'e{e}_pr_w'] for e in range(E)])
    prb = jnp.stack([params[f'e{e}_pr_b'][None, :] for e in range(E)])
    clw = jnp.stack([jnp.pad(params[f'e{e}_cl_w'], ((0, 0), (0, 118))) for e in range(E)])
    clb = jnp.stack([jnp.pad(params[f'e{e}_cl_b'], (0, 118))[None, :] for e in range(E)])

    a2 = _run_enc12(bexp_f, xs.reshape(B_PAD, 1024, 3), w1, b1, w2, b2)
    lg = _run_enc34(bexp_f, a2, w3, b3, w4, b4, prw, prb, clw, clb)

    out128 = _sc_gather(lg, slot.reshape(B))
    return out128[:, :10]


# speedup vs baseline: 1.0833x; 1.0032x over previous
import functools
import math

import jax
import jax.numpy as jnp
import numpy as np
from jax import lax
from jax.experimental import pallas as pl
from jax.experimental.pallas import tpu as pltpu
from jax.experimental.pallas import tpu_sc as plsc

E = 4
B = 1024
C = 256
BT = 32
NBLK = 40
B_PAD = NBLK * BT
NITER = 15
_F32 = jnp.float32
_I32 = jnp.int32


def _conv2d(x, w, b):
    y = jax.lax.conv_general_dilated(x, w, window_strides=(1, 1), padding='SAME',
                                     dimension_numbers=('NCHW', 'OIHW', 'NCHW'))
    return y + b[None, :, None, None]


def _avgpool(x, k):
    return jax.lax.reduce_window(x, 0.0, jax.lax.add, (1, 1, k, k), (1, 1, k, k), 'VALID') / float(k * k)


def _scorer_trunk(p, e, x):
    h = jax.nn.relu(_conv2d(x, p[f'e{e}_sc_w'], p[f'e{e}_sc_b']))
    k = h.shape[2] // 4
    h = _avgpool(h, k)
    h = h.reshape(h.shape[0], -1)
    h = jax.nn.relu(h @ p[f'e{e}_sfc_w'] + p[f'e{e}_sfc_b'])
    return h


def _scores_noisy(x, params, targets):
    Bn = x.shape[0]
    feats = [_scorer_trunk(params, e, x) for e in range(E)]
    scores = jnp.stack([(feats[e] @ params[f'e{e}_sh_w'] + params[f'e{e}_sh_b'])[:, 0]
                        for e in range(E)], axis=1)
    class_logits = jnp.stack([feats[e] @ params[f'e{e}_scl_w'] + params[f'e{e}_scl_b']
                              for e in range(E)], axis=1)
    class_probs = jax.nn.softmax(class_logits, axis=2)
    tgt = jnp.broadcast_to(targets[:, None].astype(jnp.int32), (Bn, E))
    gt_probs = jnp.take_along_axis(class_probs, tgt[:, :, None], axis=2)[:, :, 0]
    combined = scores + 1.0 * jnp.log(jnp.clip(gt_probs, 1e-9, None))
    return combined / 0.1


def _waterfall_body(sn_ref, assign_ref, slot_ref, perm_ref, bexp_ref):
    sn = sn_ref[:]
    f32, i32 = _F32, _I32

    def iota_f32(shape, dim):
        return jax.lax.broadcasted_iota(i32, shape, dim).astype(f32)

    rj = jax.lax.broadcasted_iota(i32, (B, B), 0)
    ci = jax.lax.broadcasted_iota(i32, (B, B), 1)
    ltt = (rj <= ci).astype(f32)

    iota_e = iota_f32((E, B), 0)
    iota_e1 = iota_f32((E, 1), 0)

    assign = jnp.zeros((E, B), f32)
    for it in range(NITER):
        cap = jnp.sum(assign, axis=1, keepdims=True)
        rem = 1.0 - jnp.sum(assign, axis=0, keepdims=True)
        deficit = jnp.clip(cap * (1.0 / C), 0.0, 1.0)
        s = sn - deficit
        s = jnp.where(cap >= C, -1e30, s)
        m = jnp.max(s, axis=0, keepdims=True)
        cand = jnp.where(s == m, iota_e, float(E))
        sel = jnp.min(cand, axis=0, keepdims=True)
        onehot = (iota_e == sel).astype(f32)
        want = onehot * rem
        rank = jnp.dot(want, ltt, preferred_element_type=f32)
        space = jnp.minimum(C - cap, float(2 ** it))
        take = want * (rank <= space).astype(f32)
        assign = assign + take

    cap = jnp.sum(assign, axis=1, keepdims=True)
    rem = 1.0 - jnp.sum(assign, axis=0, keepdims=True)
    mn = jnp.min(cap, axis=0, keepdims=True)
    cand = jnp.where(cap == mn, iota_e1, float(E))
    least = jnp.min(cand, axis=0, keepdims=True)
    assign = assign + (iota_e1 == least).astype(f32) * rem
    assign_ref[:] = assign

    count = jnp.sum(assign, axis=1, keepdims=True)
    pc = jnp.floor((count + (BT - 1)) * (1.0 / BT)) * BT
    slt4 = (jax.lax.broadcasted_iota(i32, (E, E), 0) >
            jax.lax.broadcasted_iota(i32, (E, E), 1)).astype(f32)
    starts = jnp.dot(slt4, pc, preferred_element_type=f32)
    rank_all = jnp.dot(assign, ltt, preferred_element_type=f32)
    slotv = jnp.sum(assign * (starts + rank_all - 1.0), axis=0, keepdims=True)
    slot_ref[:] = slotv.astype(i32)

    oh = (iota_f32((B_PAD, B), 0) == slotv).astype(f32)
    idx_col = iota_f32((B, 1), 0)
    perm = jnp.dot(oh, idx_col, preferred_element_type=f32)
    perm_ref[:] = perm.astype(i32)

    bstart = starts * (1.0 / BT)
    iota_g = iota_f32((E, NBLK), 1)
    bexp = jnp.sum((iota_g >= bstart).astype(f32), axis=0, keepdims=True) - 1.0
    bexp_ref[:] = bexp.astype(i32)


def _waterfall_route(sn_t):
    return pl.pallas_call(
        _waterfall_body,
        out_shape=(
            jax.ShapeDtypeStruct((E, B), _F32),
            jax.ShapeDtypeStruct((1, B), _I32),
            jax.ShapeDtypeStruct((B_PAD, 1), _I32),
            jax.ShapeDtypeStruct((1, NBLK), _I32),
        ),
    )(sn_t)


def _sc_gather(table, idx):
    nrows, d = idx.shape[0], table.shape[1]
    info = plsc.get_sparse_core_info()
    nw = info.num_cores * info.num_subcores
    bw = nrows // nw
    mesh = plsc.VectorSubcoreMesh(core_axis_name="c", subcore_axis_name="s")

    @functools.partial(
        pl.kernel, mesh=mesh,
        out_type=jax.ShapeDtypeStruct((nrows, d), _F32),
        scratch_types=[
            pltpu.VMEM((bw,), _I32),
            pltpu.VMEM((bw, d), _F32),
            pltpu.SemaphoreType.DMA,
        ],
    )
    def gat(table_hbm, idx_hbm, out_hbm, idx_v, rows_v, sem):
        wid = lax.axis_index("s") * info.num_cores + lax.axis_index("c")
        base = wid * bw
        pltpu.sync_copy(idx_hbm.at[pl.ds(base, bw)], idx_v)
        pltpu.async_copy(table_hbm.at[idx_v], rows_v, sem).wait()
        pltpu.sync_copy(rows_v, out_hbm.at[pl.ds(base, bw)])

    return gat(table, idx)


def _z(*shape):
    return jnp.zeros(shape, _F32)


def _conv9(xf, w, hp, cin):
    S = xf.shape[0]
    off0 = hp + 1
    nv = S - 2 * off0
    ne = nv + 2
    parts = [xf[off0 + o - 1:off0 + o - 1 + ne, :] for o in (-hp, 0, hp)]
    xcat = jnp.concatenate(parts, axis=1)
    co = w.shape[1] // 3
    p = jnp.dot(xcat, w, preferred_element_type=_F32)
    return (p[0:nv, 0:co] + p[1:1 + nv, co:2 * co] + p[2:2 + nv, 2 * co:3 * co])


def _gridmask(nv, off0, stride, hp, lo, hi):
    r = jax.lax.broadcasted_iota(_I32, (nv, 1), 0) + off0
    qt = jax.lax.rem(r, stride)
    cm = jax.lax.rem(qt, hp)
    return ((qt >= lo) & (qt <= hi) & (cm != 0) & (cm != hp - 1)).astype(_F32)


def _enc12_body(bexp_ref, in_ref, w1_ref, b1_ref, w2_ref, b2_ref, out_ref):
    G = 4
    S = G * 1156
    w1, b1 = w1_ref[0], b1_ref[0]
    w2, b2 = w2_ref[0], b2_ref[0]
    m1 = _gridmask(S - 70, 35, 1156, 34, 35, 1120)

    def grp(i, carry):
        x4 = in_ref[pl.ds(G * i, G)].reshape(G, 32, 32, 3)
        x4 = jnp.concatenate([_z(G, 32, 1, 3), x4, _z(G, 32, 1, 3)], axis=2)
        x4 = jnp.concatenate([_z(G, 1, 34, 3), x4, _z(G, 1, 34, 3)], axis=1)
        xf = x4.reshape(S, 3)
        y1 = jnp.maximum(_conv9(xf, w1, 34, 3) + b1, 0.0) * m1
        y1f = jnp.concatenate([_z(35, 64), y1, _z(35, 64)], axis=0)
        y2 = jnp.maximum(_conv9(y1f, w2, 34, 64) + b2, 0.0) * m1
        y2f = jnp.concatenate([_z(35, 64), y2, _z(35, 64)], axis=0)
        g = y2f.reshape(G, 34, 34, 64)[:, 1:33, 1:33, :].reshape(G, 16, 2, 16, 2, 64)
        p = jnp.max(jnp.max(g, axis=4), axis=2)
        p = jnp.concatenate([_z(G, 16, 1, 64), p, _z(G, 16, 1, 64)], axis=2)
        p = jnp.concatenate([_z(G, 1, 18, 64), p, _z(G, 1, 18, 64)], axis=1)
        out_ref[pl.ds(G * i, G)] = p.reshape(G, 324, 64)
        return carry

    jax.lax.fori_loop(0, BT // G, grp, 0)


def _enc34_body(bexp_ref, in_ref, w3_ref, b3_ref, w4_ref, b4_ref,
                prw_ref, prb_ref, clw_ref, clb_ref, out_ref, fe_ref):
    G = 4
    S3 = G * 324
    S4 = G * 100
    w3, b3 = w3_ref[0], b3_ref[0]
    w4, b4 = w4_ref[0], b4_ref[0]
    m3 = _gridmask(S3 - 38, 19, 324, 18, 19, 304)
    m4 = _gridmask(S4 - 22, 11, 100, 10, 11, 88)

    def grp(i, carry):
        a4 = in_ref[pl.ds(G * i, G)].reshape(S3, 64)
        y3 = jnp.maximum(_conv9(a4, w3, 18, 64) + b3, 0.0) * m3
        y3f = jnp.concatenate([_z(19, 128), y3, _z(19, 128)], axis=0)
        g = y3f.reshape(G, 18, 18, 128)[:, 1:17, 1:17, :].reshape(G, 8, 2, 8, 2, 128)
        p = jnp.max(jnp.max(g, axis=4), axis=2)
        p = jnp.concatenate([_z(G, 8, 1, 128), p, _z(G, 8, 1, 128)], axis=2)
        p = jnp.concatenate([_z(G, 1, 10, 128), p, _z(G, 1, 10, 128)], axis=1)
        y4 = jnp.maximum(_conv9(p.reshape(S4, 128), w4, 10, 128) + b4, 0.0) * m4
        for k in range(G):
            fe = jnp.sum(y4[100 * k:100 * k + 78, :], axis=0, keepdims=True) * (1.0 / 64.0)
            fe_ref[pl.ds(G * i + k, 1), :] = fe
        return carry

    jax.lax.fori_loop(0, BT // G, grp, 0)
    feats = fe_ref[:]
    z = jnp.dot(feats, prw_ref[0], preferred_element_type=_F32) + prb_ref[0]
    out_ref[:] = jnp.dot(z, clw_ref[0], preferred_element_type=_F32) + clb_ref[0]


def _expert_spec(k, o):
    return pl.BlockSpec((1, k, o), lambda g, b: (b[g], 0, 0))


def _tok_spec(s, c):
    return pl.BlockSpec((BT, s, c), lambda g, b: (g, 0, 0))


def _run_enc12(bexp, xs, w1, b1, w2, b2):
    gs = pltpu.PrefetchScalarGridSpec(
        num_scalar_prefetch=1, grid=(NBLK,),
        in_specs=[_tok_spec(1024, 3), _expert_spec(9, 192), _expert_spec(1, 64),
                  _expert_spec(192, 192), _expert_spec(1, 64)],
        out_specs=_tok_spec(324, 64),
    )
    return pl.pallas_call(_enc12_body, grid_spec=gs,
                          out_shape=jax.ShapeDtypeStruct((B_PAD, 324, 64), _F32),
                          )(bexp, xs, w1, b1, w2, b2)


def _run_enc34(bexp, a, w3, b3, w4, b4, prw, prb, clw, clb):
    gs = pltpu.PrefetchScalarGridSpec(
        num_scalar_prefetch=1, grid=(NBLK,),
        in_specs=[_tok_spec(324, 64), _expert_spec(192, 384), _expert_spec(1, 128),
                  _expert_spec(384, 768), _expert_spec(1, 256),
                  _expert_spec(256, 256), _expert_spec(1, 256),
                  _expert_spec(256, 128), _expert_spec(1, 128)],
        out_specs=pl.BlockSpec((BT, 128), lambda g, b: (g, 0)),
        scratch_shapes=[pltpu.VMEM((BT, 256), _F32)],
    )
    return pl.pallas_call(_enc34_body, grid_spec=gs,
                          out_shape=jax.ShapeDtypeStruct((B_PAD, 128), _F32),
                          )(bexp, a, w3, b3, w4, b4, prw, prb, clw, clb)


def _fold_conv(params, layer, cin, cout):
    s = 1.0 / np.sqrt(1.0 + 1e-5)
    ws, bs = [], []
    for e in range(E):
        wv = params[f'e{e}_c{layer}_w']
        bv = params[f'e{e}_c{layer}_b']
        g = params[f'e{e}_bn{layer}_g'] * s
        bb = params[f'e{e}_bn{layer}_b']
        wim = jnp.transpose(wv, (3, 2, 1, 0)) * g[None, None, None, :]
        wd = wim.reshape(3, 3 * cin, cout)
        ws.append(jnp.concatenate([wd[0], wd[1], wd[2]], axis=1))
        bs.append((bv * g + bb)[None, :])
    return jnp.stack(ws), jnp.stack(bs)


def kernel(x, params, targets):
    sn = _scores_noisy(x, params, targets)
    assign, slot, perm, bexp = _waterfall_route(sn.T)
    del assign

    x2d = jnp.transpose(x, (0, 2, 3, 1)).reshape(B, 32 * 32 * 3)
    xs = _sc_gather(x2d, perm.reshape(B_PAD))

    bexp_f = bexp.reshape(NBLK)
    w1, b1 = _fold_conv(params, 1, 3, 64)
    w2, b2 = _fold_conv(params, 2, 64, 64)
    w3, b3 = _fold_conv(params, 3, 64, 128)
    w4, b4 = _fold_conv(params, 4, 128, 256)
    prw = jnp.stack([params[f'e{e}_pr_w'] for e in range(E)])
    prb = jnp.stack([params[f'e{e}_pr_b'][None, :] for e in range(E)])
    clw = jnp.stack([jnp.pad(params[f'e{e}_cl_w'], ((0, 0), (0, 118))) for e in range(E)])
    clb = jnp.stack([jnp.pad(params[f'e{e}_cl_b'], (0, 118))[None, :] for e in range(E)])

    a2 = _run_enc12(bexp_f, xs.reshape(B_PAD, 1024, 3), w1, b1, w2, b2)
    lg = _run_enc34(bexp_f, a2, w3, b3, w4, b4, prw, prb, clw, clb)

    out128 = _sc_gather(lg, slot.reshape(B))
    return out128[:, :10]
